# Initial kernel scaffold; baseline (speedup 1.0000x reference)
import functools

import jax
import jax.numpy as jnp
from jax import lax
from jax.experimental import pallas as pl
from jax.experimental.pallas import tpu as pltpu
from jax.experimental.pallas import tpu_sc as plsc

_N = 10000
_D = 128
_E = 320000
_OUT = 16

_NC = 2
_NS = 16
_NW = _NC * _NS
_CHUNK = 128
_CHUNKS = 79
_EPAD = _NW * _CHUNKS * _CHUNK
_RPT = 640
_NACC = _NS * _RPT


def _make_agg():
    mesh = plsc.VectorSubcoreMesh(core_axis_name="c", subcore_axis_name="s")

    @functools.partial(
        pl.kernel,
        out_type=jax.ShapeDtypeStruct((_NC, _NACC, _D), jnp.float32),
        mesh=mesh,
        scratch_types=[
            pltpu.VMEM((_CHUNKS, _CHUNK), jnp.int32),
            pltpu.VMEM((_CHUNKS, _CHUNK), jnp.int32),
            pltpu.VMEM((_CHUNK, _D), jnp.float32),
            pltpu.VMEM_SHARED((_NACC, _D), jnp.float32),
            pltpu.SemaphoreType.DMA,
        ],
    )
    def agg(x_hbm, src_hbm, dst_hbm, zero_hbm, out_hbm, src_v, dst_v, rows_v,
            acc, sem):
        c = lax.axis_index("c")
        s = lax.axis_index("s")
        w = s * _NC + c
        pltpu.sync_copy(zero_hbm, acc.at[pl.ds(s * _RPT, _RPT)])
        pltpu.sync_copy(src_hbm.at[w], src_v)
        pltpu.sync_copy(dst_hbm.at[w], dst_v)
        plsc.subcore_barrier()

        def body(j, carry):
            pltpu.async_copy(x_hbm.at[src_v.at[j]], rows_v, sem).wait()
            pltpu.sync_copy(rows_v, acc.at[dst_v.at[j]], add=True)
            return carry

        lax.fori_loop(0, _CHUNKS, body, 0)
        plsc.subcore_barrier()
        pltpu.sync_copy(acc.at[pl.ds(s * _RPT, _RPT)],
                        out_hbm.at[c, pl.ds(s * _RPT, _RPT)])

    return agg


_agg_call = _make_agg()

_ROWS_BLK = 500
_GRID = _N // _ROWS_BLK


def _mlp_body(x_ref, a0_ref, a1_ref, w1_ref, b1_ref, w2_ref, b2_ref,
              g_ref, be_ref, rm_ref, rv_ref, o_ref):
    h = x_ref[...] + a0_ref[0] + a1_ref[0]
    t = jnp.dot(h, w1_ref[...], preferred_element_type=jnp.float32)
    t = jnp.maximum(t + b1_ref[...], 0.0)
    t = jnp.dot(t, w2_ref[...], preferred_element_type=jnp.float32)
    t = jnp.maximum(t + b2_ref[...], 0.0)
    inv = lax.rsqrt(rv_ref[...] + 1e-5)
    o_ref[...] = (t - rm_ref[...]) * inv * g_ref[...] + be_ref[...]


def _mlp_call(x, agg, w1, b1, w2, b2, g, be, rm, rv):
    vec = pl.BlockSpec((1, _D), lambda i: (0, 0))
    return pl.pallas_call(
        _mlp_body,
        grid=(_GRID,),
        in_specs=[
            pl.BlockSpec((_ROWS_BLK, _D), lambda i: (i, 0)),
            pl.BlockSpec((1, _ROWS_BLK, _D), lambda i: (0, i, 0)),
            pl.BlockSpec((1, _ROWS_BLK, _D), lambda i: (1, i, 0)),
            pl.BlockSpec((_D, _D), lambda i: (0, 0)),
            vec,
            pl.BlockSpec((_D, _D), lambda i: (0, 0)),
            vec, vec, vec, vec, vec,
        ],
        out_specs=pl.BlockSpec((_ROWS_BLK, _D), lambda i: (i, 0)),
        out_shape=jax.ShapeDtypeStruct((_N, _D), jnp.float32),
    )(x, agg, agg, w1, b1.reshape(1, _D), w2, b2.reshape(1, _D),
      g.reshape(1, _D), be.reshape(1, _D), rm.reshape(1, _D),
      rv.reshape(1, _D))


def _jk_body(h1_ref, h2_ref, h3_ref, l1_ref, b1_ref, l2_ref, b2_ref, o_ref):
    t = jnp.dot(h1_ref[...], l1_ref[0:_D], preferred_element_type=jnp.float32)
    t += jnp.dot(h2_ref[...], l1_ref[_D:2 * _D],
                 preferred_element_type=jnp.float32)
    t += jnp.dot(h3_ref[...], l1_ref[2 * _D:3 * _D],
                 preferred_element_type=jnp.float32)
    t = jnp.maximum(t + b1_ref[...], 0.0)
    o_ref[...] = jnp.dot(t, l2_ref[...],
                         preferred_element_type=jnp.float32) + b2_ref[...]


def _jk_call(h1, h2, h3, lin1_w, lin1_b, lin2_w, lin2_b):
    rows = pl.BlockSpec((_ROWS_BLK, _D), lambda i: (i, 0))
    return pl.pallas_call(
        _jk_body,
        grid=(_GRID,),
        in_specs=[
            rows, rows, rows,
            pl.BlockSpec((3 * _D, _D), lambda i: (0, 0)),
            pl.BlockSpec((1, _D), lambda i: (0, 0)),
            pl.BlockSpec((_D, _OUT), lambda i: (0, 0)),
            pl.BlockSpec((1, _OUT), lambda i: (0, 0)),
        ],
        out_specs=pl.BlockSpec((_ROWS_BLK, _OUT), lambda i: (i, 0)),
        out_shape=jax.ShapeDtypeStruct((_N, _OUT), jnp.float32),
    )(h1, h2, h3, lin1_w, lin1_b.reshape(1, _D), lin2_w,
      lin2_b.reshape(1, _OUT))


def kernel(x, edge_index, batch,
           w1_0, b1_0, w2_0, b2_0, g_0, be_0, rm_0, rv_0,
           w1_1, b1_1, w2_1, b2_1, g_1, be_1, rm_1, rv_1,
           w1_2, b1_2, w2_2, b2_2, g_2, be_2, rm_2, rv_2,
           lin1_w, lin1_b, lin2_w, lin2_b):
    del batch
    pad = _EPAD - _E
    src = jnp.concatenate([edge_index[0], jnp.zeros((pad,), jnp.int32)])
    dst = jnp.concatenate([edge_index[1], jnp.full((pad,), _N, jnp.int32)])
    src = src.reshape(_NW, _CHUNKS, _CHUNK)
    dst = dst.reshape(_NW, _CHUNKS, _CHUNK)
    zero = jnp.zeros((_RPT, _D), jnp.float32)

    params = [
        (w1_0, b1_0, w2_0, b2_0, g_0, be_0, rm_0, rv_0),
        (w1_1, b1_1, w2_1, b2_1, g_1, be_1, rm_1, rv_1),
        (w1_2, b1_2, w2_2, b2_2, g_2, be_2, rm_2, rv_2),
    ]
    h = x
    hs = []
    for p in params:
        agg = _agg_call(h, src, dst, zero)
        h = _mlp_call(h, agg, *p)
        hs.append(h)
    return _jk_call(hs[0], hs[1], hs[2], lin1_w, lin1_b, lin2_w, lin2_b)


# speedup vs baseline: 4.2360x; 4.2360x over previous
import functools

import jax
import jax.numpy as jnp
from jax import lax
from jax.experimental import pallas as pl
from jax.experimental.pallas import tpu as pltpu
from jax.experimental.pallas import tpu_sc as plsc

_N = 10000
_D = 128
_E = 320000
_OUT = 16

_NC = 2
_NS = 16
_NW = _NC * _NS
_CHUNK = 128
_CHUNKS = 79
_EPAD = _NW * _CHUNKS * _CHUNK
_RPT = 640
_NACC = _NS * _RPT


def _make_agg():
    mesh = plsc.VectorSubcoreMesh(core_axis_name="c", subcore_axis_name="s")

    @functools.partial(
        pl.kernel,
        out_type=jax.ShapeDtypeStruct((_NC, _NACC, _D), jnp.float32),
        mesh=mesh,
        scratch_types=[
            pltpu.VMEM((_CHUNKS, _CHUNK), jnp.int32),
            pltpu.VMEM((_CHUNKS, _CHUNK), jnp.int32),
            pltpu.VMEM((_CHUNK, _D), jnp.float32),
            pltpu.VMEM_SHARED((_NACC, _D), jnp.float32),
            pltpu.SemaphoreType.DMA,
        ],
    )
    def agg(x_hbm, src_hbm, dst_hbm, zero_hbm, out_hbm, src_v, dst_v, rows_v,
            acc, sem):
        c = lax.axis_index("c")
        s = lax.axis_index("s")
        w = s * _NC + c
        pltpu.sync_copy(zero_hbm, acc.at[pl.ds(s * _RPT, _RPT)])
        pltpu.sync_copy(src_hbm.at[w], src_v)
        pltpu.sync_copy(dst_hbm.at[w], dst_v)
        plsc.subcore_barrier()

        def body(j, carry):
            pltpu.async_copy(x_hbm.at[src_v.at[j]], rows_v, sem).wait()
            pltpu.sync_copy(rows_v, acc.at[dst_v.at[j]], add=True)
            return carry

        lax.fori_loop(0, _CHUNKS, body, 0)
        plsc.subcore_barrier()
        pltpu.sync_copy(acc.at[pl.ds(s * _RPT, _RPT)],
                        out_hbm.at[c, pl.ds(s * _RPT, _RPT)])

    return agg


_agg_call = _make_agg()

_ROWS_BLK = 400
_GRID = _N // _ROWS_BLK


def _mlp_body(x_ref, a0_ref, a1_ref, w1_ref, b1_ref, w2_ref, b2_ref,
              g_ref, be_ref, rm_ref, rv_ref, o_ref):
    h = x_ref[...] + a0_ref[0] + a1_ref[0]
    t = jnp.dot(h, w1_ref[...], preferred_element_type=jnp.float32)
    t = jnp.maximum(t + b1_ref[...], 0.0)
    t = jnp.dot(t, w2_ref[...], preferred_element_type=jnp.float32)
    t = jnp.maximum(t + b2_ref[...], 0.0)
    inv = lax.rsqrt(rv_ref[...] + 1e-5)
    o_ref[...] = (t - rm_ref[...]) * inv * g_ref[...] + be_ref[...]


def _mlp_call(x, agg, w1, b1, w2, b2, g, be, rm, rv):
    vec = pl.BlockSpec((1, _D), lambda i: (0, 0))
    return pl.pallas_call(
        _mlp_body,
        grid=(_GRID,),
        in_specs=[
            pl.BlockSpec((_ROWS_BLK, _D), lambda i: (i, 0)),
            pl.BlockSpec((1, _ROWS_BLK, _D), lambda i: (0, i, 0)),
            pl.BlockSpec((1, _ROWS_BLK, _D), lambda i: (1, i, 0)),
            pl.BlockSpec((_D, _D), lambda i: (0, 0)),
            vec,
            pl.BlockSpec((_D, _D), lambda i: (0, 0)),
            vec, vec, vec, vec, vec,
        ],
        out_specs=pl.BlockSpec((_ROWS_BLK, _D), lambda i: (i, 0)),
        out_shape=jax.ShapeDtypeStruct((_N, _D), jnp.float32),
    )(x, agg, agg, w1, b1.reshape(1, _D), w2, b2.reshape(1, _D),
      g.reshape(1, _D), be.reshape(1, _D), rm.reshape(1, _D),
      rv.reshape(1, _D))


def _jk_body(h1_ref, h2_ref, h3_ref, l1_ref, b1_ref, l2_ref, b2_ref, o_ref):
    t = jnp.dot(h1_ref[...], l1_ref[0:_D], preferred_element_type=jnp.float32)
    t += jnp.dot(h2_ref[...], l1_ref[_D:2 * _D],
                 preferred_element_type=jnp.float32)
    t += jnp.dot(h3_ref[...], l1_ref[2 * _D:3 * _D],
                 preferred_element_type=jnp.float32)
    t = jnp.maximum(t + b1_ref[...], 0.0)
    o_ref[...] = jnp.dot(t, l2_ref[...],
                         preferred_element_type=jnp.float32) + b2_ref[...]


def _jk_call(h1, h2, h3, lin1_w, lin1_b, lin2_w, lin2_b):
    rows = pl.BlockSpec((_ROWS_BLK, _D), lambda i: (i, 0))
    return pl.pallas_call(
        _jk_body,
        grid=(_GRID,),
        in_specs=[
            rows, rows, rows,
            pl.BlockSpec((3 * _D, _D), lambda i: (0, 0)),
            pl.BlockSpec((1, _D), lambda i: (0, 0)),
            pl.BlockSpec((_D, _OUT), lambda i: (0, 0)),
            pl.BlockSpec((1, _OUT), lambda i: (0, 0)),
        ],
        out_specs=pl.BlockSpec((_ROWS_BLK, _OUT), lambda i: (i, 0)),
        out_shape=jax.ShapeDtypeStruct((_N, _OUT), jnp.float32),
    )(h1, h2, h3, lin1_w, lin1_b.reshape(1, _D), lin2_w,
      lin2_b.reshape(1, _OUT))


def kernel(x, edge_index, batch,
           w1_0, b1_0, w2_0, b2_0, g_0, be_0, rm_0, rv_0,
           w1_1, b1_1, w2_1, b2_1, g_1, be_1, rm_1, rv_1,
           w1_2, b1_2, w2_2, b2_2, g_2, be_2, rm_2, rv_2,
           lin1_w, lin1_b, lin2_w, lin2_b):
    del batch
    pad = _EPAD - _E
    src = jnp.concatenate([edge_index[0], jnp.zeros((pad,), jnp.int32)])
    dst = jnp.concatenate([edge_index[1], jnp.full((pad,), _N, jnp.int32)])
    src = src.reshape(_NW, _CHUNKS, _CHUNK)
    dst = dst.reshape(_NW, _CHUNKS, _CHUNK)
    zero = jnp.zeros((_RPT, _D), jnp.float32)

    params = [
        (w1_0, b1_0, w2_0, b2_0, g_0, be_0, rm_0, rv_0),
        (w1_1, b1_1, w2_1, b2_1, g_1, be_1, rm_1, rv_1),
        (w1_2, b1_2, w2_2, b2_2, g_2, be_2, rm_2, rv_2),
    ]
    h = x
    hs = []
    for p in params:
        agg = _agg_call(h, src, dst, zero)
        h = _mlp_call(h, agg, *p)
        hs.append(h)
    return _jk_call(hs[0], hs[1], hs[2], lin1_w, lin1_b, lin2_w, lin2_b)


# speedup vs baseline: 10.8881x; 2.5704x over previous
import functools

import jax
import jax.numpy as jnp
from jax import lax
from jax.experimental import pallas as pl
from jax.experimental.pallas import tpu as pltpu
from jax.experimental.pallas import tpu_sc as plsc

_N = 10000
_D = 128
_E = 320000
_OUT = 16

_NC = 2
_NS = 16
_NW = _NC * _NS
_CHUNK = 128
_CHUNKS = 80
_PH = 40
_EPAD = _NW * _CHUNKS * _CHUNK
_RPT = 632
_NACC = _NS * _RPT


def _make_agg():
    mesh = plsc.VectorSubcoreMesh(core_axis_name="c", subcore_axis_name="s")

    @functools.partial(
        pl.kernel,
        out_type=jax.ShapeDtypeStruct((_NC, _NACC, _D), jnp.float32),
        mesh=mesh,
        scratch_types=[
            pltpu.VMEM((_PH * _CHUNK,), jnp.int32),
            pltpu.VMEM((_CHUNKS, _CHUNK), jnp.int32),
            pltpu.VMEM((_CHUNK, _D), jnp.float32),
            pltpu.VMEM((_CHUNK, _D), jnp.float32),
            pltpu.VMEM_SHARED((_NACC, _D), jnp.float32),
            pltpu.SemaphoreType.DMA,
            pltpu.SemaphoreType.DMA,
        ],
    )
    def agg(x_hbm, src_hbm, dst_hbm, zero_hbm, out_hbm, src_v, dst_v, rows_a,
            rows_b, acc, sem_a, sem_b):
        c = lax.axis_index("c")
        s = lax.axis_index("s")
        w = s * _NC + c
        pltpu.sync_copy(zero_hbm, acc.at[pl.ds(s * _RPT, _RPT)])
        pltpu.sync_copy(dst_hbm.at[w], dst_v)
        plsc.subcore_barrier()

        def run_phase(ph):
            pltpu.sync_copy(src_hbm.at[w, pl.ds(ph * _PH * _CHUNK,
                                                _PH * _CHUNK)], src_v)

            def gather(l, buf, sem):
                pltpu.async_copy(
                    x_hbm.at[src_v.at[pl.ds(l * _CHUNK, _CHUNK)]], buf, sem)

            def wait(l, buf, sem):
                pltpu.make_async_copy(
                    x_hbm.at[src_v.at[pl.ds(l * _CHUNK, _CHUNK)]],
                    buf, sem).wait()

            def scat(l, buf):
                pltpu.sync_copy(buf, acc.at[dst_v.at[ph * _PH + l]], add=True)

            gather(0, rows_a, sem_a)

            def body(t, carry):
                l = 2 * t
                gather(l + 1, rows_b, sem_b)
                wait(l, rows_a, sem_a)
                scat(l, rows_a)
                gather(l + 2, rows_a, sem_a)
                wait(l + 1, rows_b, sem_b)
                scat(l + 1, rows_b)
                return carry

            lax.fori_loop(0, _PH // 2 - 1, body, 0)
            gather(_PH - 1, rows_b, sem_b)
            wait(_PH - 2, rows_a, sem_a)
            scat(_PH - 2, rows_a)
            wait(_PH - 1, rows_b, sem_b)
            scat(_PH - 1, rows_b)

        run_phase(0)
        run_phase(1)
        plsc.subcore_barrier()
        pltpu.sync_copy(acc.at[pl.ds(s * _RPT, _RPT)],
                        out_hbm.at[c, pl.ds(s * _RPT, _RPT)])

    return agg


_agg_call = _make_agg()

_ROWS_BLK = 400
_GRID = _N // _ROWS_BLK


def _mlp_body(x_ref, a0_ref, a1_ref, w1_ref, b1_ref, w2_ref, b2_ref,
              g_ref, be_ref, rm_ref, rv_ref, o_ref):
    h = x_ref[...] + a0_ref[0] + a1_ref[0]
    t = jnp.dot(h, w1_ref[...], preferred_element_type=jnp.float32)
    t = jnp.maximum(t + b1_ref[...], 0.0)
    t = jnp.dot(t, w2_ref[...], preferred_element_type=jnp.float32)
    t = jnp.maximum(t + b2_ref[...], 0.0)
    inv = lax.rsqrt(rv_ref[...] + 1e-5)
    o_ref[...] = (t - rm_ref[...]) * inv * g_ref[...] + be_ref[...]


def _mlp_call(x, agg, w1, b1, w2, b2, g, be, rm, rv):
    vec = pl.BlockSpec((1, _D), lambda i: (0, 0))
    return pl.pallas_call(
        _mlp_body,
        grid=(_GRID,),
        in_specs=[
            pl.BlockSpec((_ROWS_BLK, _D), lambda i: (i, 0)),
            pl.BlockSpec((1, _ROWS_BLK, _D), lambda i: (0, i, 0)),
            pl.BlockSpec((1, _ROWS_BLK, _D), lambda i: (1, i, 0)),
            pl.BlockSpec((_D, _D), lambda i: (0, 0)),
            vec,
            pl.BlockSpec((_D, _D), lambda i: (0, 0)),
            vec, vec, vec, vec, vec,
        ],
        out_specs=pl.BlockSpec((_ROWS_BLK, _D), lambda i: (i, 0)),
        out_shape=jax.ShapeDtypeStruct((_N, _D), jnp.float32),
    )(x, agg, agg, w1, b1.reshape(1, _D), w2, b2.reshape(1, _D),
      g.reshape(1, _D), be.reshape(1, _D), rm.reshape(1, _D),
      rv.reshape(1, _D))


def _jk_body(h1_ref, h2_ref, h3_ref, l1_ref, b1_ref, l2_ref, b2_ref, o_ref):
    t = jnp.dot(h1_ref[...], l1_ref[0:_D], preferred_element_type=jnp.float32)
    t += jnp.dot(h2_ref[...], l1_ref[_D:2 * _D],
                 preferred_element_type=jnp.float32)
    t += jnp.dot(h3_ref[...], l1_ref[2 * _D:3 * _D],
                 preferred_element_type=jnp.float32)
    t = jnp.maximum(t + b1_ref[...], 0.0)
    o_ref[...] = jnp.dot(t, l2_ref[...],
                         preferred_element_type=jnp.float32) + b2_ref[...]


def _jk_call(h1, h2, h3, lin1_w, lin1_b, lin2_w, lin2_b):
    rows = pl.BlockSpec((_ROWS_BLK, _D), lambda i: (i, 0))
    return pl.pallas_call(
        _jk_body,
        grid=(_GRID,),
        in_specs=[
            rows, rows, rows,
            pl.BlockSpec((3 * _D, _D), lambda i: (0, 0)),
            pl.BlockSpec((1, _D), lambda i: (0, 0)),
            pl.BlockSpec((_D, _OUT), lambda i: (0, 0)),
            pl.BlockSpec((1, _OUT), lambda i: (0, 0)),
        ],
        out_specs=pl.BlockSpec((_ROWS_BLK, _OUT), lambda i: (i, 0)),
        out_shape=jax.ShapeDtypeStruct((_N, _OUT), jnp.float32),
    )(h1, h2, h3, lin1_w, lin1_b.reshape(1, _D), lin2_w,
      lin2_b.reshape(1, _OUT))


def kernel(x, edge_index, batch,
           w1_0, b1_0, w2_0, b2_0, g_0, be_0, rm_0, rv_0,
           w1_1, b1_1, w2_1, b2_1, g_1, be_1, rm_1, rv_1,
           w1_2, b1_2, w2_2, b2_2, g_2, be_2, rm_2, rv_2,
           lin1_w, lin1_b, lin2_w, lin2_b):
    del batch
    pad = _EPAD - _E
    pad_ar = jnp.arange(pad, dtype=jnp.int32)
    src = jnp.concatenate([edge_index[0], pad_ar % _N])
    dst = jnp.concatenate([edge_index[1], _N + pad_ar % (_NACC - _N)])
    src = src.reshape(_NW, _CHUNKS * _CHUNK)
    dst = dst.reshape(_NW, _CHUNKS, _CHUNK)
    zero = jnp.zeros((_RPT, _D), jnp.float32)

    params = [
        (w1_0, b1_0, w2_0, b2_0, g_0, be_0, rm_0, rv_0),
        (w1_1, b1_1, w2_1, b2_1, g_1, be_1, rm_1, rv_1),
        (w1_2, b1_2, w2_2, b2_2, g_2, be_2, rm_2, rv_2),
    ]
    h = x
    hs = []
    for p in params:
        agg = _agg_call(h, src, dst, zero)
        h = _mlp_call(h, agg, *p)
        hs.append(h)
    return _jk_call(hs[0], hs[1], hs[2], lin1_w, lin1_b, lin2_w, lin2_b)


# speedup vs baseline: 11.8462x; 1.0880x over previous
import functools

import jax
import jax.numpy as jnp
from jax import lax
from jax.experimental import pallas as pl
from jax.experimental.pallas import tpu as pltpu
from jax.experimental.pallas import tpu_sc as plsc

_N = 10000
_D = 128
_E = 320000
_OUT = 16

_NC = 2
_NS = 16
_NW = _NC * _NS
_CHUNK = 100
_CHUNKS = 100
_PH = 50
_RPT = 632
_NACC = _NS * _RPT


def _make_agg():
    mesh = plsc.VectorSubcoreMesh(core_axis_name="c", subcore_axis_name="s")

    @functools.partial(
        pl.kernel,
        out_type=jax.ShapeDtypeStruct((_NC, _NACC, _D), jnp.float32),
        mesh=mesh,
        scratch_types=[
            pltpu.VMEM((_PH, _CHUNK), jnp.int32),
            pltpu.VMEM((_CHUNKS, _CHUNK), jnp.int32),
            pltpu.VMEM((_CHUNK, _D), jnp.float32),
            pltpu.VMEM((_CHUNK, _D), jnp.float32),
            pltpu.VMEM_SHARED((_NACC, _D), jnp.float32),
            pltpu.SemaphoreType.DMA,
            pltpu.SemaphoreType.DMA,
        ],
    )
    def agg(x_hbm, er_hbm, erp_hbm, zero_hbm, out_hbm, src_v, dst_v,
            rows_a, rows_b, acc, sem_a, sem_b):
        c = lax.axis_index("c")
        s = lax.axis_index("s")
        w = s * _NC + c
        pltpu.sync_copy(zero_hbm, acc.at[pl.ds(s * _RPT, _RPT)])
        pltpu.sync_copy(er_hbm.at[1, w], dst_v)
        plsc.subcore_barrier()

        def run_phase(ph):
            pltpu.sync_copy(erp_hbm.at[0, w, ph], src_v)

            def gather(l, buf, sem):
                pltpu.async_copy(x_hbm.at[src_v.at[l]], buf, sem)

            def wait(l, buf, sem):
                pltpu.make_async_copy(x_hbm.at[src_v.at[l]], buf, sem).wait()

            def scat(l, buf):
                pltpu.sync_copy(buf, acc.at[dst_v.at[ph * _PH + l]], add=True)

            gather(0, rows_a, sem_a)

            def body(t, carry):
                l = 2 * t
                gather(l + 1, rows_b, sem_b)
                wait(l, rows_a, sem_a)
                scat(l, rows_a)
                gather(l + 2, rows_a, sem_a)
                wait(l + 1, rows_b, sem_b)
                scat(l + 1, rows_b)
                return carry

            lax.fori_loop(0, _PH // 2 - 1, body, 0)
            gather(_PH - 1, rows_b, sem_b)
            wait(_PH - 2, rows_a, sem_a)
            scat(_PH - 2, rows_a)
            wait(_PH - 1, rows_b, sem_b)
            scat(_PH - 1, rows_b)

        run_phase(0)
        run_phase(1)
        plsc.subcore_barrier()
        pltpu.sync_copy(acc.at[pl.ds(s * _RPT, _RPT)],
                        out_hbm.at[c, pl.ds(s * _RPT, _RPT)])

    return agg


_agg_call = _make_agg()

_ROWS_BLK = 2000
_GRID = _N // _ROWS_BLK


def _mlp_body(x_ref, a0_ref, a1_ref, w1_ref, b1_ref, w2_ref, b2_ref,
              g_ref, be_ref, rm_ref, rv_ref, o_ref):
    h = x_ref[...] + a0_ref[0] + a1_ref[0]
    t = jnp.dot(h, w1_ref[...], preferred_element_type=jnp.float32)
    t = jnp.maximum(t + b1_ref[...], 0.0)
    t = jnp.dot(t, w2_ref[...], preferred_element_type=jnp.float32)
    t = jnp.maximum(t + b2_ref[...], 0.0)
    inv = lax.rsqrt(rv_ref[...] + 1e-5)
    o_ref[...] = (t - rm_ref[...]) * inv * g_ref[...] + be_ref[...]


def _mlp_call(x, agg, w1, b1, w2, b2, g, be, rm, rv):
    vec = pl.BlockSpec((1, _D), lambda i: (0, 0))
    return pl.pallas_call(
        _mlp_body,
        grid=(_GRID,),
        in_specs=[
            pl.BlockSpec((_ROWS_BLK, _D), lambda i: (i, 0)),
            pl.BlockSpec((1, _ROWS_BLK, _D), lambda i: (0, i, 0)),
            pl.BlockSpec((1, _ROWS_BLK, _D), lambda i: (1, i, 0)),
            pl.BlockSpec((_D, _D), lambda i: (0, 0)),
            vec,
            pl.BlockSpec((_D, _D), lambda i: (0, 0)),
            vec, vec, vec, vec, vec,
        ],
        out_specs=pl.BlockSpec((_ROWS_BLK, _D), lambda i: (i, 0)),
        out_shape=jax.ShapeDtypeStruct((_N, _D), jnp.float32),
    )(x, agg, agg, w1, b1.reshape(1, _D), w2, b2.reshape(1, _D),
      g.reshape(1, _D), be.reshape(1, _D), rm.reshape(1, _D),
      rv.reshape(1, _D))


def _jkpart_body(h1_ref, h2_ref, l1_ref, o_ref):
    t = jnp.dot(h1_ref[...], l1_ref[0:_D], preferred_element_type=jnp.float32)
    t += jnp.dot(h2_ref[...], l1_ref[_D:2 * _D],
                 preferred_element_type=jnp.float32)
    o_ref[...] = t


def _jkpart_call(h1, h2, lin1_w):
    rows = pl.BlockSpec((_ROWS_BLK, _D), lambda i: (i, 0))
    return pl.pallas_call(
        _jkpart_body,
        grid=(_GRID,),
        in_specs=[rows, rows, pl.BlockSpec((3 * _D, _D), lambda i: (0, 0))],
        out_specs=rows,
        out_shape=jax.ShapeDtypeStruct((_N, _D), jnp.float32),
    )(h1, h2, lin1_w)


def _jkfin_body(p_ref, h3_ref, l1_ref, b1_ref, l2_ref, b2_ref, o_ref):
    t = p_ref[...] + jnp.dot(h3_ref[...], l1_ref[2 * _D:3 * _D],
                             preferred_element_type=jnp.float32)
    t = jnp.maximum(t + b1_ref[...], 0.0)
    o_ref[...] = jnp.dot(t, l2_ref[...],
                         preferred_element_type=jnp.float32) + b2_ref[...]


def _jkfin_call(p, h3, lin1_w, lin1_b, lin2_w, lin2_b):
    rows = pl.BlockSpec((_ROWS_BLK, _D), lambda i: (i, 0))
    return pl.pallas_call(
        _jkfin_body,
        grid=(_GRID,),
        in_specs=[
            rows, rows,
            pl.BlockSpec((3 * _D, _D), lambda i: (0, 0)),
            pl.BlockSpec((1, _D), lambda i: (0, 0)),
            pl.BlockSpec((_D, _OUT), lambda i: (0, 0)),
            pl.BlockSpec((1, _OUT), lambda i: (0, 0)),
        ],
        out_specs=pl.BlockSpec((_ROWS_BLK, _OUT), lambda i: (i, 0)),
        out_shape=jax.ShapeDtypeStruct((_N, _OUT), jnp.float32),
    )(p, h3, lin1_w, lin1_b.reshape(1, _D), lin2_w, lin2_b.reshape(1, _OUT))


def kernel(x, edge_index, batch,
           w1_0, b1_0, w2_0, b2_0, g_0, be_0, rm_0, rv_0,
           w1_1, b1_1, w2_1, b2_1, g_1, be_1, rm_1, rv_1,
           w1_2, b1_2, w2_2, b2_2, g_2, be_2, rm_2, rv_2,
           lin1_w, lin1_b, lin2_w, lin2_b):
    del batch
    er = edge_index.reshape(2, _NW, _CHUNKS, _CHUNK)
    erp = edge_index.reshape(2, _NW, _CHUNKS // _PH, _PH, _CHUNK)
    zero = jnp.zeros((_RPT, _D), jnp.float32)

    params = [
        (w1_0, b1_0, w2_0, b2_0, g_0, be_0, rm_0, rv_0),
        (w1_1, b1_1, w2_1, b2_1, g_1, be_1, rm_1, rv_1),
        (w1_2, b1_2, w2_2, b2_2, g_2, be_2, rm_2, rv_2),
    ]
    h = x
    hs = []
    for p in params:
        agg = _agg_call(h, er, erp, zero)
        h = _mlp_call(h, agg, *p)
        hs.append(h)
    part = _jkpart_call(hs[0], hs[1], lin1_w)
    return _jkfin_call(part, hs[2], lin1_w, lin1_b, lin2_w, lin2_b)


# speedup vs baseline: 12.1658x; 1.0270x over previous
import functools

import jax
import jax.numpy as jnp
from jax import lax
from jax.experimental import pallas as pl
from jax.experimental.pallas import tpu as pltpu
from jax.experimental.pallas import tpu_sc as plsc

_N = 10000
_D = 128
_E = 320000
_OUT = 16

_NC = 2
_NS = 16
_NW = _NC * _NS
_CHUNK = 100
_CHUNKS = 100
_PH = 50
_RPT = 632
_NACC = _NS * _RPT


def _make_agg():
    mesh = plsc.VectorSubcoreMesh(core_axis_name="c", subcore_axis_name="s")

    @functools.partial(
        pl.kernel,
        out_type=jax.ShapeDtypeStruct((_NC, _NACC, _D), jnp.float32),
        mesh=mesh,
        scratch_types=[
            pltpu.VMEM((_PH, _CHUNK), jnp.int32),
            pltpu.VMEM((2, _PH, _CHUNK), jnp.int32),
            pltpu.VMEM((_CHUNK, _D), jnp.float32),
            pltpu.VMEM((_CHUNK, _D), jnp.float32),
            pltpu.VMEM_SHARED((_NACC, _D), jnp.float32),
            pltpu.SemaphoreType.DMA,
            pltpu.SemaphoreType.DMA,
        ],
    )
    def agg(x_hbm, er_hbm, zero_hbm, out_hbm, src_v, dst_v,
            rows_a, rows_b, acc, sem_a, sem_b):
        c = lax.axis_index("c")
        s = lax.axis_index("s")
        w = s * _NC + c
        pltpu.sync_copy(zero_hbm, acc.at[pl.ds(s * _RPT, _RPT)])
        pltpu.sync_copy(er_hbm.at[1, w, 0], dst_v.at[0])
        pltpu.sync_copy(er_hbm.at[1, w, 1], dst_v.at[1])
        plsc.subcore_barrier()

        def run_phase(ph):
            pltpu.sync_copy(er_hbm.at[0, w, ph], src_v)

            def gather(l, buf, sem):
                pltpu.async_copy(x_hbm.at[src_v.at[l]], buf, sem)

            def wait(l, buf, sem):
                pltpu.make_async_copy(x_hbm.at[src_v.at[l]], buf, sem).wait()

            def scat(l, buf):
                pltpu.sync_copy(buf, acc.at[dst_v.at[ph, l]], add=True)

            gather(0, rows_a, sem_a)

            def body(t, carry):
                l = 2 * t
                gather(l + 1, rows_b, sem_b)
                wait(l, rows_a, sem_a)
                scat(l, rows_a)
                gather(l + 2, rows_a, sem_a)
                wait(l + 1, rows_b, sem_b)
                scat(l + 1, rows_b)
                return carry

            lax.fori_loop(0, _PH // 2 - 1, body, 0)
            gather(_PH - 1, rows_b, sem_b)
            wait(_PH - 2, rows_a, sem_a)
            scat(_PH - 2, rows_a)
            wait(_PH - 1, rows_b, sem_b)
            scat(_PH - 1, rows_b)

        run_phase(0)
        run_phase(1)
        plsc.subcore_barrier()
        pltpu.sync_copy(acc.at[pl.ds(s * _RPT, _RPT)],
                        out_hbm.at[c, pl.ds(s * _RPT, _RPT)])

    return agg


_agg_call = _make_agg()

_ROWS_BLK = 2000
_GRID = _N // _ROWS_BLK


def _mlp_body(x_ref, a0_ref, a1_ref, w1_ref, b1_ref, w2_ref, b2_ref,
              g_ref, be_ref, rm_ref, rv_ref, o_ref):
    h = x_ref[...] + a0_ref[0] + a1_ref[0]
    t = jnp.dot(h, w1_ref[...], preferred_element_type=jnp.float32)
    t = jnp.maximum(t + b1_ref[...], 0.0)
    t = jnp.dot(t, w2_ref[...], preferred_element_type=jnp.float32)
    t = jnp.maximum(t + b2_ref[...], 0.0)
    inv = lax.rsqrt(rv_ref[...] + 1e-5)
    o_ref[...] = (t - rm_ref[...]) * inv * g_ref[...] + be_ref[...]


def _mlp_call(x, agg, w1, b1, w2, b2, g, be, rm, rv):
    vec = pl.BlockSpec((1, _D), lambda i: (0, 0))
    return pl.pallas_call(
        _mlp_body,
        grid=(_GRID,),
        in_specs=[
            pl.BlockSpec((_ROWS_BLK, _D), lambda i: (i, 0)),
            pl.BlockSpec((1, _ROWS_BLK, _D), lambda i: (0, i, 0)),
            pl.BlockSpec((1, _ROWS_BLK, _D), lambda i: (1, i, 0)),
            pl.BlockSpec((_D, _D), lambda i: (0, 0)),
            vec,
            pl.BlockSpec((_D, _D), lambda i: (0, 0)),
            vec, vec, vec, vec, vec,
        ],
        out_specs=pl.BlockSpec((_ROWS_BLK, _D), lambda i: (i, 0)),
        out_shape=jax.ShapeDtypeStruct((_N, _D), jnp.float32),
    )(x, agg, agg, w1, b1.reshape(1, _D), w2, b2.reshape(1, _D),
      g.reshape(1, _D), be.reshape(1, _D), rm.reshape(1, _D),
      rv.reshape(1, _D))


def _jkpart_body(h1_ref, h2_ref, l1_ref, o_ref):
    t = jnp.dot(h1_ref[...], l1_ref[0:_D], preferred_element_type=jnp.float32)
    t += jnp.dot(h2_ref[...], l1_ref[_D:2 * _D],
                 preferred_element_type=jnp.float32)
    o_ref[...] = t


def _jkpart_call(h1, h2, lin1_w):
    rows = pl.BlockSpec((_ROWS_BLK, _D), lambda i: (i, 0))
    return pl.pallas_call(
        _jkpart_body,
        grid=(_GRID,),
        in_specs=[rows, rows, pl.BlockSpec((3 * _D, _D), lambda i: (0, 0))],
        out_specs=rows,
        out_shape=jax.ShapeDtypeStruct((_N, _D), jnp.float32),
    )(h1, h2, lin1_w)


def _jkfin_body(x_ref, a0_ref, a1_ref, w1_ref, b1_ref, w2_ref, b2_ref,
                g_ref, be_ref, rm_ref, rv_ref, p_ref, l1_ref, jb1_ref,
                l2_ref, jb2_ref, o_ref):
    h = x_ref[...] + a0_ref[0] + a1_ref[0]
    t = jnp.dot(h, w1_ref[...], preferred_element_type=jnp.float32)
    t = jnp.maximum(t + b1_ref[...], 0.0)
    t = jnp.dot(t, w2_ref[...], preferred_element_type=jnp.float32)
    t = jnp.maximum(t + b2_ref[...], 0.0)
    inv = lax.rsqrt(rv_ref[...] + 1e-5)
    h3 = (t - rm_ref[...]) * inv * g_ref[...] + be_ref[...]
    t = p_ref[...] + jnp.dot(h3, l1_ref[2 * _D:3 * _D],
                             preferred_element_type=jnp.float32)
    t = jnp.maximum(t + jb1_ref[...], 0.0)
    o_ref[...] = jnp.dot(t, l2_ref[...],
                         preferred_element_type=jnp.float32) + jb2_ref[...]


def _jkfin_call(x, agg, w1, b1, w2, b2, g, be, rm, rv, p,
                lin1_w, lin1_b, lin2_w, lin2_b):
    rows = pl.BlockSpec((_ROWS_BLK, _D), lambda i: (i, 0))
    vec = pl.BlockSpec((1, _D), lambda i: (0, 0))
    return pl.pallas_call(
        _jkfin_body,
        grid=(_GRID,),
        in_specs=[
            rows,
            pl.BlockSpec((1, _ROWS_BLK, _D), lambda i: (0, i, 0)),
            pl.BlockSpec((1, _ROWS_BLK, _D), lambda i: (1, i, 0)),
            pl.BlockSpec((_D, _D), lambda i: (0, 0)),
            vec,
            pl.BlockSpec((_D, _D), lambda i: (0, 0)),
            vec, vec, vec, vec, vec,
            rows,
            pl.BlockSpec((3 * _D, _D), lambda i: (0, 0)),
            vec,
            pl.BlockSpec((_D, _OUT), lambda i: (0, 0)),
            pl.BlockSpec((1, _OUT), lambda i: (0, 0)),
        ],
        out_specs=pl.BlockSpec((_ROWS_BLK, _OUT), lambda i: (i, 0)),
        out_shape=jax.ShapeDtypeStruct((_N, _OUT), jnp.float32),
    )(x, agg, agg, w1, b1.reshape(1, _D), w2, b2.reshape(1, _D),
      g.reshape(1, _D), be.reshape(1, _D), rm.reshape(1, _D),
      rv.reshape(1, _D), p, lin1_w, lin1_b.reshape(1, _D), lin2_w,
      lin2_b.reshape(1, _OUT))


def kernel(x, edge_index, batch,
           w1_0, b1_0, w2_0, b2_0, g_0, be_0, rm_0, rv_0,
           w1_1, b1_1, w2_1, b2_1, g_1, be_1, rm_1, rv_1,
           w1_2, b1_2, w2_2, b2_2, g_2, be_2, rm_2, rv_2,
           lin1_w, lin1_b, lin2_w, lin2_b):
    del batch
    er = edge_index.reshape(2, _NW, _CHUNKS // _PH, _PH, _CHUNK)
    zero = jnp.zeros((_RPT, _D), jnp.float32)

    params = [
        (w1_0, b1_0, w2_0, b2_0, g_0, be_0, rm_0, rv_0),
        (w1_1, b1_1, w2_1, b2_1, g_1, be_1, rm_1, rv_1),
        (w1_2, b1_2, w2_2, b2_2, g_2, be_2, rm_2, rv_2),
    ]
    h = x
    hs = []
    for p in params[:2]:
        agg = _agg_call(h, er, zero)
        h = _mlp_call(h, agg, *p)
        hs.append(h)
    part = _jkpart_call(hs[0], hs[1], lin1_w)
    agg = _agg_call(hs[1], er, zero)
    return _jkfin_call(hs[1], agg, *params[2], part,
                       lin1_w, lin1_b, lin2_w, lin2_b)


# speedup vs baseline: 12.3477x; 1.0149x over previous
import functools

import jax
import jax.numpy as jnp
from jax import lax
from jax.experimental import pallas as pl
from jax.experimental.pallas import tpu as pltpu
from jax.experimental.pallas import tpu_sc as plsc

_N = 10000
_D = 128
_E = 320000
_OUT = 16

_NC = 2
_NS = 16
_NW = _NC * _NS
_CHUNK = 100
_CHUNKS = 100
_PH = 50
_RPT = 640
_NACC = _NS * _RPT


def _make_agg():
    mesh = plsc.VectorSubcoreMesh(core_axis_name="c", subcore_axis_name="s")

    @functools.partial(
        pl.kernel,
        out_type=jax.ShapeDtypeStruct((_NC, _NACC, _D), jnp.bfloat16),
        mesh=mesh,
        compiler_params=pltpu.CompilerParams(use_tc_tiling_on_sc=False),
        scratch_types=[
            pltpu.VMEM((_PH, _CHUNK), jnp.int32),
            pltpu.VMEM((2, _PH, _CHUNK), jnp.int32),
            pltpu.VMEM((_CHUNK, _D), jnp.bfloat16),
            pltpu.VMEM((_CHUNK, _D), jnp.bfloat16),
            pltpu.VMEM_SHARED((_NACC, _D), jnp.bfloat16),
            pltpu.SemaphoreType.DMA,
            pltpu.SemaphoreType.DMA,
        ],
    )
    def agg(x_hbm, er_hbm, zero_hbm, out_hbm, src_v, dst_v,
            rows_a, rows_b, acc, sem_a, sem_b):
        c = lax.axis_index("c")
        s = lax.axis_index("s")
        w = s * _NC + c
        pltpu.sync_copy(zero_hbm, acc.at[pl.ds(s * _RPT, _RPT)])
        pltpu.sync_copy(er_hbm.at[1, w, 0], dst_v.at[0])
        pltpu.sync_copy(er_hbm.at[1, w, 1], dst_v.at[1])
        plsc.subcore_barrier()

        def run_phase(ph):
            pltpu.sync_copy(er_hbm.at[0, w, ph], src_v)

            def gather(l, buf, sem):
                pltpu.async_copy(x_hbm.at[src_v.at[l]], buf, sem)

            def wait(l, buf, sem):
                pltpu.make_async_copy(x_hbm.at[src_v.at[l]], buf, sem).wait()

            def scat(l, buf):
                pltpu.sync_copy(buf, acc.at[dst_v.at[ph, l]], add=True)

            gather(0, rows_a, sem_a)

            def body(t, carry):
                l = 2 * t
                gather(l + 1, rows_b, sem_b)
                wait(l, rows_a, sem_a)
                scat(l, rows_a)
                gather(l + 2, rows_a, sem_a)
                wait(l + 1, rows_b, sem_b)
                scat(l + 1, rows_b)
                return carry

            lax.fori_loop(0, _PH // 2 - 1, body, 0)
            gather(_PH - 1, rows_b, sem_b)
            wait(_PH - 2, rows_a, sem_a)
            scat(_PH - 2, rows_a)
            wait(_PH - 1, rows_b, sem_b)
            scat(_PH - 1, rows_b)

        run_phase(0)
        run_phase(1)
        plsc.subcore_barrier()
        pltpu.sync_copy(acc.at[pl.ds(s * _RPT, _RPT)],
                        out_hbm.at[c, pl.ds(s * _RPT, _RPT)])

    return agg


_agg_call = _make_agg()

_ROWS_BLK = 2000
_GRID = _N // _ROWS_BLK


def _mlp_body(x_ref, a0_ref, a1_ref, w1_ref, b1_ref, w2_ref, b2_ref,
              g_ref, be_ref, rm_ref, rv_ref, o_ref, ob_ref):
    h = x_ref[...] + (a0_ref[0].astype(jnp.float32)
                      + a1_ref[0].astype(jnp.float32))
    t = jnp.dot(h, w1_ref[...], preferred_element_type=jnp.float32)
    t = jnp.maximum(t + b1_ref[...], 0.0)
    t = jnp.dot(t, w2_ref[...], preferred_element_type=jnp.float32)
    t = jnp.maximum(t + b2_ref[...], 0.0)
    inv = lax.rsqrt(rv_ref[...] + 1e-5)
    hn = (t - rm_ref[...]) * inv * g_ref[...] + be_ref[...]
    o_ref[...] = hn
    ob_ref[...] = hn.astype(jnp.bfloat16)


def _mlp_call(x, agg, w1, b1, w2, b2, g, be, rm, rv):
    vec = pl.BlockSpec((1, _D), lambda i: (0, 0))
    return pl.pallas_call(
        _mlp_body,
        grid=(_GRID,),
        in_specs=[
            pl.BlockSpec((_ROWS_BLK, _D), lambda i: (i, 0)),
            pl.BlockSpec((1, _ROWS_BLK, _D), lambda i: (0, i, 0)),
            pl.BlockSpec((1, _ROWS_BLK, _D), lambda i: (1, i, 0)),
            pl.BlockSpec((_D, _D), lambda i: (0, 0)),
            vec,
            pl.BlockSpec((_D, _D), lambda i: (0, 0)),
            vec, vec, vec, vec, vec,
        ],
        out_specs=[pl.BlockSpec((_ROWS_BLK, _D), lambda i: (i, 0)),
                   pl.BlockSpec((_ROWS_BLK, _D), lambda i: (i, 0))],
        out_shape=[jax.ShapeDtypeStruct((_N, _D), jnp.float32),
                   jax.ShapeDtypeStruct((_N, _D), jnp.bfloat16)],
    )(x, agg, agg, w1, b1.reshape(1, _D), w2, b2.reshape(1, _D),
      g.reshape(1, _D), be.reshape(1, _D), rm.reshape(1, _D),
      rv.reshape(1, _D))


def _jkpart_body(h1_ref, h2_ref, l1_ref, o_ref):
    t = jnp.dot(h1_ref[...], l1_ref[0:_D], preferred_element_type=jnp.float32)
    t += jnp.dot(h2_ref[...], l1_ref[_D:2 * _D],
                 preferred_element_type=jnp.float32)
    o_ref[...] = t


def _jkpart_call(h1, h2, lin1_w):
    rows = pl.BlockSpec((_ROWS_BLK, _D), lambda i: (i, 0))
    return pl.pallas_call(
        _jkpart_body,
        grid=(_GRID,),
        in_specs=[rows, rows, pl.BlockSpec((3 * _D, _D), lambda i: (0, 0))],
        out_specs=rows,
        out_shape=jax.ShapeDtypeStruct((_N, _D), jnp.float32),
    )(h1, h2, lin1_w)


def _jkfin_body(x_ref, a0_ref, a1_ref, w1_ref, b1_ref, w2_ref, b2_ref,
                g_ref, be_ref, rm_ref, rv_ref, p_ref, l1_ref, jb1_ref,
                l2_ref, jb2_ref, o_ref):
    h = x_ref[...] + (a0_ref[0].astype(jnp.float32)
                      + a1_ref[0].astype(jnp.float32))
    t = jnp.dot(h, w1_ref[...], preferred_element_type=jnp.float32)
    t = jnp.maximum(t + b1_ref[...], 0.0)
    t = jnp.dot(t, w2_ref[...], preferred_element_type=jnp.float32)
    t = jnp.maximum(t + b2_ref[...], 0.0)
    inv = lax.rsqrt(rv_ref[...] + 1e-5)
    h3 = (t - rm_ref[...]) * inv * g_ref[...] + be_ref[...]
    t = p_ref[...] + jnp.dot(h3, l1_ref[2 * _D:3 * _D],
                             preferred_element_type=jnp.float32)
    t = jnp.maximum(t + jb1_ref[...], 0.0)
    o_ref[...] = jnp.dot(t, l2_ref[...],
                         preferred_element_type=jnp.float32) + jb2_ref[...]


def _jkfin_call(x, agg, w1, b1, w2, b2, g, be, rm, rv, p,
                lin1_w, lin1_b, lin2_w, lin2_b):
    rows = pl.BlockSpec((_ROWS_BLK, _D), lambda i: (i, 0))
    vec = pl.BlockSpec((1, _D), lambda i: (0, 0))
    return pl.pallas_call(
        _jkfin_body,
        grid=(_GRID,),
        in_specs=[
            rows,
            pl.BlockSpec((1, _ROWS_BLK, _D), lambda i: (0, i, 0)),
            pl.BlockSpec((1, _ROWS_BLK, _D), lambda i: (1, i, 0)),
            pl.BlockSpec((_D, _D), lambda i: (0, 0)),
            vec,
            pl.BlockSpec((_D, _D), lambda i: (0, 0)),
            vec, vec, vec, vec, vec,
            rows,
            pl.BlockSpec((3 * _D, _D), lambda i: (0, 0)),
            vec,
            pl.BlockSpec((_D, _OUT), lambda i: (0, 0)),
            pl.BlockSpec((1, _OUT), lambda i: (0, 0)),
        ],
        out_specs=pl.BlockSpec((_ROWS_BLK, _OUT), lambda i: (i, 0)),
        out_shape=jax.ShapeDtypeStruct((_N, _OUT), jnp.float32),
    )(x, agg, agg, w1, b1.reshape(1, _D), w2, b2.reshape(1, _D),
      g.reshape(1, _D), be.reshape(1, _D), rm.reshape(1, _D),
      rv.reshape(1, _D), p, lin1_w, lin1_b.reshape(1, _D), lin2_w,
      lin2_b.reshape(1, _OUT))


def kernel(x, edge_index, batch,
           w1_0, b1_0, w2_0, b2_0, g_0, be_0, rm_0, rv_0,
           w1_1, b1_1, w2_1, b2_1, g_1, be_1, rm_1, rv_1,
           w1_2, b1_2, w2_2, b2_2, g_2, be_2, rm_2, rv_2,
           lin1_w, lin1_b, lin2_w, lin2_b):
    del batch
    er = edge_index.reshape(2, _NW, _CHUNKS // _PH, _PH, _CHUNK)
    zero = jnp.zeros((_RPT, _D), jnp.bfloat16)

    params = [
        (w1_0, b1_0, w2_0, b2_0, g_0, be_0, rm_0, rv_0),
        (w1_1, b1_1, w2_1, b2_1, g_1, be_1, rm_1, rv_1),
        (w1_2, b1_2, w2_2, b2_2, g_2, be_2, rm_2, rv_2),
    ]
    h, h_bf = x, x.astype(jnp.bfloat16)
    hs = []
    for p in params[:2]:
        agg = _agg_call(h_bf, er, zero)
        h, h_bf = _mlp_call(h, agg, *p)
        hs.append(h)
    part = _jkpart_call(hs[0], hs[1], lin1_w)
    agg = _agg_call(h_bf, er, zero)
    return _jkfin_call(hs[1], agg, *params[2], part,
                       lin1_w, lin1_b, lin2_w, lin2_b)


# speedup vs baseline: 14.9982x; 1.2147x over previous
import functools

import jax
import jax.numpy as jnp
from jax import lax
from jax.experimental import pallas as pl
from jax.experimental.pallas import tpu as pltpu
from jax.experimental.pallas import tpu_sc as plsc

_N = 10000
_D = 128
_E = 320000
_OUT = 16

_NC = 2
_NS = 16
_NW = _NC * _NS
_CHUNK = 100
_CHUNKS = 100
_PH = 50
_RPT = 640
_NACC = _NS * _RPT


def _make_agg():
    mesh = plsc.VectorSubcoreMesh(core_axis_name="c", subcore_axis_name="s")

    @functools.partial(
        pl.kernel,
        out_type=jax.ShapeDtypeStruct((_NC, _NACC, _D), jnp.bfloat16),
        mesh=mesh,
        compiler_params=pltpu.CompilerParams(use_tc_tiling_on_sc=False),
        scratch_types=[
            pltpu.VMEM((_CHUNKS, _CHUNK), jnp.int32),
            pltpu.VMEM((_CHUNKS, _CHUNK), jnp.int32),
            pltpu.VMEM((_CHUNK, _D), jnp.bfloat16),
            pltpu.VMEM((_CHUNK, _D), jnp.bfloat16),
            pltpu.VMEM((_CHUNK, _D), jnp.bfloat16),
            pltpu.VMEM((_CHUNK, _D), jnp.bfloat16),
            pltpu.VMEM_SHARED((_NACC, _D), jnp.bfloat16),
            pltpu.SemaphoreType.DMA,
            pltpu.SemaphoreType.DMA,
            pltpu.SemaphoreType.DMA,
            pltpu.SemaphoreType.DMA,
        ],
    )
    def agg(x_hbm, er_hbm, zero_hbm, out_hbm, src_v, dst_v,
            r0, r1, r2, r3, acc, s0, s1, s2, s3):
        c = lax.axis_index("c")
        s = lax.axis_index("s")
        w = s * _NC + c
        pltpu.sync_copy(zero_hbm, acc.at[pl.ds(s * _RPT, _RPT)])
        pltpu.sync_copy(er_hbm.at[0, w], src_v)
        pltpu.sync_copy(er_hbm.at[1, w], dst_v)
        plsc.subcore_barrier()

        bufs = ((r0, s0), (r1, s1), (r2, s2), (r3, s3))

        def gather(l, buf, sem):
            pltpu.async_copy(x_hbm.at[src_v.at[l]], buf, sem)

        def wait(l, buf, sem):
            pltpu.make_async_copy(x_hbm.at[src_v.at[l]], buf, sem).wait()

        def scat(l, buf):
            pltpu.sync_copy(buf, acc.at[dst_v.at[l]], add=True)

        for k in range(4):
            gather(k, *bufs[k])

        def body(t, carry):
            l0 = 4 * t
            for k in range(4):
                wait(l0 + k, *bufs[k])
                scat(l0 + k, bufs[k][0])
                gather(l0 + k + 4, *bufs[k])
            return carry

        lax.fori_loop(0, _CHUNKS // 4 - 1, body, 0)
        for k in range(4):
            l = _CHUNKS - 4 + k
            wait(l, *bufs[k])
            scat(l, bufs[k][0])
        plsc.subcore_barrier()
        pltpu.sync_copy(acc.at[pl.ds(s * _RPT, _RPT)],
                        out_hbm.at[c, pl.ds(s * _RPT, _RPT)])

    return agg


_agg_call = _make_agg()

_ROWS_BLK = 2000
_GRID = _N // _ROWS_BLK


def _mlp_body(x_ref, a0_ref, a1_ref, w1_ref, b1_ref, w2_ref, b2_ref,
              g_ref, be_ref, rm_ref, rv_ref, o_ref, ob_ref):
    h = x_ref[...] + (a0_ref[0].astype(jnp.float32)
                      + a1_ref[0].astype(jnp.float32))
    t = jnp.dot(h, w1_ref[...], preferred_element_type=jnp.float32)
    t = jnp.maximum(t + b1_ref[...], 0.0)
    t = jnp.dot(t, w2_ref[...], preferred_element_type=jnp.float32)
    t = jnp.maximum(t + b2_ref[...], 0.0)
    inv = lax.rsqrt(rv_ref[...] + 1e-5)
    hn = (t - rm_ref[...]) * inv * g_ref[...] + be_ref[...]
    o_ref[...] = hn
    ob_ref[...] = hn.astype(jnp.bfloat16)


def _mlp_call(x, agg, w1, b1, w2, b2, g, be, rm, rv):
    vec = pl.BlockSpec((1, _D), lambda i: (0, 0))
    return pl.pallas_call(
        _mlp_body,
        grid=(_GRID,),
        in_specs=[
            pl.BlockSpec((_ROWS_BLK, _D), lambda i: (i, 0)),
            pl.BlockSpec((1, _ROWS_BLK, _D), lambda i: (0, i, 0)),
            pl.BlockSpec((1, _ROWS_BLK, _D), lambda i: (1, i, 0)),
            pl.BlockSpec((_D, _D), lambda i: (0, 0)),
            vec,
            pl.BlockSpec((_D, _D), lambda i: (0, 0)),
            vec, vec, vec, vec, vec,
        ],
        out_specs=[pl.BlockSpec((_ROWS_BLK, _D), lambda i: (i, 0)),
                   pl.BlockSpec((_ROWS_BLK, _D), lambda i: (i, 0))],
        out_shape=[jax.ShapeDtypeStruct((_N, _D), jnp.float32),
                   jax.ShapeDtypeStruct((_N, _D), jnp.bfloat16)],
    )(x, agg, agg, w1, b1.reshape(1, _D), w2, b2.reshape(1, _D),
      g.reshape(1, _D), be.reshape(1, _D), rm.reshape(1, _D),
      rv.reshape(1, _D))


def _jkpart_body(h1_ref, h2_ref, l1_ref, o_ref):
    t = jnp.dot(h1_ref[...], l1_ref[0:_D], preferred_element_type=jnp.float32)
    t += jnp.dot(h2_ref[...], l1_ref[_D:2 * _D],
                 preferred_element_type=jnp.float32)
    o_ref[...] = t


def _jkpart_call(h1, h2, lin1_w):
    rows = pl.BlockSpec((_ROWS_BLK, _D), lambda i: (i, 0))
    return pl.pallas_call(
        _jkpart_body,
        grid=(_GRID,),
        in_specs=[rows, rows, pl.BlockSpec((3 * _D, _D), lambda i: (0, 0))],
        out_specs=rows,
        out_shape=jax.ShapeDtypeStruct((_N, _D), jnp.float32),
    )(h1, h2, lin1_w)


def _jkfin_body(x_ref, a0_ref, a1_ref, w1_ref, b1_ref, w2_ref, b2_ref,
                g_ref, be_ref, rm_ref, rv_ref, p_ref, l1_ref, jb1_ref,
                l2_ref, jb2_ref, o_ref):
    h = x_ref[...] + (a0_ref[0].astype(jnp.float32)
                      + a1_ref[0].astype(jnp.float32))
    t = jnp.dot(h, w1_ref[...], preferred_element_type=jnp.float32)
    t = jnp.maximum(t + b1_ref[...], 0.0)
    t = jnp.dot(t, w2_ref[...], preferred_element_type=jnp.float32)
    t = jnp.maximum(t + b2_ref[...], 0.0)
    inv = lax.rsqrt(rv_ref[...] + 1e-5)
    h3 = (t - rm_ref[...]) * inv * g_ref[...] + be_ref[...]
    t = p_ref[...] + jnp.dot(h3, l1_ref[2 * _D:3 * _D],
                             preferred_element_type=jnp.float32)
    t = jnp.maximum(t + jb1_ref[...], 0.0)
    o_ref[...] = jnp.dot(t, l2_ref[...],
                         preferred_element_type=jnp.float32) + jb2_ref[...]


def _jkfin_call(x, agg, w1, b1, w2, b2, g, be, rm, rv, p,
                lin1_w, lin1_b, lin2_w, lin2_b):
    rows = pl.BlockSpec((_ROWS_BLK, _D), lambda i: (i, 0))
    vec = pl.BlockSpec((1, _D), lambda i: (0, 0))
    return pl.pallas_call(
        _jkfin_body,
        grid=(_GRID,),
        in_specs=[
            rows,
            pl.BlockSpec((1, _ROWS_BLK, _D), lambda i: (0, i, 0)),
            pl.BlockSpec((1, _ROWS_BLK, _D), lambda i: (1, i, 0)),
            pl.BlockSpec((_D, _D), lambda i: (0, 0)),
            vec,
            pl.BlockSpec((_D, _D), lambda i: (0, 0)),
            vec, vec, vec, vec, vec,
            rows,
            pl.BlockSpec((3 * _D, _D), lambda i: (0, 0)),
            vec,
            pl.BlockSpec((_D, _OUT), lambda i: (0, 0)),
            pl.BlockSpec((1, _OUT), lambda i: (0, 0)),
        ],
        out_specs=pl.BlockSpec((_ROWS_BLK, _OUT), lambda i: (i, 0)),
        out_shape=jax.ShapeDtypeStruct((_N, _OUT), jnp.float32),
    )(x, agg, agg, w1, b1.reshape(1, _D), w2, b2.reshape(1, _D),
      g.reshape(1, _D), be.reshape(1, _D), rm.reshape(1, _D),
      rv.reshape(1, _D), p, lin1_w, lin1_b.reshape(1, _D), lin2_w,
      lin2_b.reshape(1, _OUT))


def kernel(x, edge_index, batch,
           w1_0, b1_0, w2_0, b2_0, g_0, be_0, rm_0, rv_0,
           w1_1, b1_1, w2_1, b2_1, g_1, be_1, rm_1, rv_1,
           w1_2, b1_2, w2_2, b2_2, g_2, be_2, rm_2, rv_2,
           lin1_w, lin1_b, lin2_w, lin2_b):
    del batch
    er = edge_index.reshape(2, _NW, _CHUNKS, _CHUNK)
    zero = jnp.zeros((_RPT, _D), jnp.bfloat16)

    params = [
        (w1_0, b1_0, w2_0, b2_0, g_0, be_0, rm_0, rv_0),
        (w1_1, b1_1, w2_1, b2_1, g_1, be_1, rm_1, rv_1),
        (w1_2, b1_2, w2_2, b2_2, g_2, be_2, rm_2, rv_2),
    ]
    h, h_bf = x, x.astype(jnp.bfloat16)
    hs = []
    for p in params[:2]:
        agg = _agg_call(h_bf, er, zero)
        h, h_bf = _mlp_call(h, agg, *p)
        hs.append(h)
    part = _jkpart_call(hs[0], hs[1], lin1_w)
    agg = _agg_call(h_bf, er, zero)
    return _jkfin_call(hs[1], agg, *params[2], part,
                       lin1_w, lin1_b, lin2_w, lin2_b)


# speedup vs baseline: 15.0477x; 1.0033x over previous
import functools

import jax
import jax.numpy as jnp
from jax import lax
from jax.experimental import pallas as pl
from jax.experimental.pallas import tpu as pltpu
from jax.experimental.pallas import tpu_sc as plsc

_N = 10000
_D = 128
_E = 320000
_OUT = 16

_NC = 2
_NS = 16
_NW = _NC * _NS
_CHUNK = 100
_CHUNKS = 100
_PH = 50
_RPT = 640
_NACC = _NS * _RPT


def _make_agg():
    mesh = plsc.VectorSubcoreMesh(core_axis_name="c", subcore_axis_name="s")

    @functools.partial(
        pl.kernel,
        out_type=jax.ShapeDtypeStruct((_NC, _NACC, _D), jnp.bfloat16),
        mesh=mesh,
        compiler_params=pltpu.CompilerParams(use_tc_tiling_on_sc=False),
        scratch_types=[
            pltpu.VMEM((_CHUNKS, _CHUNK), jnp.int32),
            pltpu.VMEM((_CHUNKS, _CHUNK), jnp.int32),
            pltpu.VMEM((_CHUNK, _D), jnp.bfloat16),
            pltpu.VMEM((_CHUNK, _D), jnp.bfloat16),
            pltpu.VMEM((_CHUNK, _D), jnp.bfloat16),
            pltpu.VMEM((_CHUNK, _D), jnp.bfloat16),
            pltpu.VMEM((_CHUNK, _D), jnp.bfloat16),
            pltpu.VMEM((_CHUNK, _D), jnp.bfloat16),
            pltpu.VMEM_SHARED((_NACC, _D), jnp.bfloat16),
            pltpu.SemaphoreType.DMA,
            pltpu.SemaphoreType.DMA,
            pltpu.SemaphoreType.DMA,
            pltpu.SemaphoreType.DMA,
            pltpu.SemaphoreType.DMA,
            pltpu.SemaphoreType.DMA,
        ],
    )
    def agg(x_hbm, er_hbm, zero_hbm, out_hbm, src_v, dst_v,
            r0, r1, r2, r3, r4, r5, acc, s0, s1, s2, s3, s4, s5):
        c = lax.axis_index("c")
        s = lax.axis_index("s")
        w = s * _NC + c
        pltpu.sync_copy(zero_hbm, acc.at[pl.ds(s * _RPT, _RPT)])
        pltpu.sync_copy(er_hbm.at[0, w], src_v)
        pltpu.sync_copy(er_hbm.at[1, w], dst_v)
        plsc.subcore_barrier()

        bufs = ((r0, s0), (r1, s1), (r2, s2), (r3, s3),
                (r4, s4), (r5, s5))

        def gather(l, buf, sem):
            pltpu.async_copy(x_hbm.at[src_v.at[l]], buf, sem)

        def wait(l, buf, sem):
            pltpu.make_async_copy(x_hbm.at[src_v.at[l]], buf, sem).wait()

        def scat(l, buf):
            pltpu.sync_copy(buf, acc.at[dst_v.at[l]], add=True)

        _R = 6
        for k in range(_R):
            gather(k, *bufs[k])

        def body(t, carry):
            l0 = _R * t
            for k in range(_R):
                wait(l0 + k, *bufs[k])
                scat(l0 + k, bufs[k][0])
                gather(l0 + k + _R, *bufs[k])
            return carry

        lax.fori_loop(0, _CHUNKS // _R - 1, body, 0)
        base = _R * (_CHUNKS // _R - 1)
        for l in range(base, _CHUNKS):
            wait(l, *bufs[l % _R])
            scat(l, bufs[l % _R][0])
            if l + _R < _CHUNKS:
                gather(l + _R, *bufs[(l + _R) % _R])
        plsc.subcore_barrier()
        pltpu.sync_copy(acc.at[pl.ds(s * _RPT, _RPT)],
                        out_hbm.at[c, pl.ds(s * _RPT, _RPT)])

    return agg


_agg_call = _make_agg()

_ROWS_BLK = 2000
_GRID = _N // _ROWS_BLK


def _mlp_body(x_ref, a0_ref, a1_ref, w1_ref, b1_ref, w2_ref, b2_ref,
              g_ref, be_ref, rm_ref, rv_ref, o_ref, ob_ref):
    h = x_ref[...] + (a0_ref[0].astype(jnp.float32)
                      + a1_ref[0].astype(jnp.float32))
    t = jnp.dot(h, w1_ref[...], preferred_element_type=jnp.float32)
    t = jnp.maximum(t + b1_ref[...], 0.0)
    t = jnp.dot(t, w2_ref[...], preferred_element_type=jnp.float32)
    t = jnp.maximum(t + b2_ref[...], 0.0)
    inv = lax.rsqrt(rv_ref[...] + 1e-5)
    hn = (t - rm_ref[...]) * inv * g_ref[...] + be_ref[...]
    o_ref[...] = hn
    ob_ref[...] = hn.astype(jnp.bfloat16)


def _mlp_call(x, agg, w1, b1, w2, b2, g, be, rm, rv):
    vec = pl.BlockSpec((1, _D), lambda i: (0, 0))
    return pl.pallas_call(
        _mlp_body,
        grid=(_GRID,),
        in_specs=[
            pl.BlockSpec((_ROWS_BLK, _D), lambda i: (i, 0)),
            pl.BlockSpec((1, _ROWS_BLK, _D), lambda i: (0, i, 0)),
            pl.BlockSpec((1, _ROWS_BLK, _D), lambda i: (1, i, 0)),
            pl.BlockSpec((_D, _D), lambda i: (0, 0)),
            vec,
            pl.BlockSpec((_D, _D), lambda i: (0, 0)),
            vec, vec, vec, vec, vec,
        ],
        out_specs=[pl.BlockSpec((_ROWS_BLK, _D), lambda i: (i, 0)),
                   pl.BlockSpec((_ROWS_BLK, _D), lambda i: (i, 0))],
        out_shape=[jax.ShapeDtypeStruct((_N, _D), jnp.float32),
                   jax.ShapeDtypeStruct((_N, _D), jnp.bfloat16)],
    )(x, agg, agg, w1, b1.reshape(1, _D), w2, b2.reshape(1, _D),
      g.reshape(1, _D), be.reshape(1, _D), rm.reshape(1, _D),
      rv.reshape(1, _D))


def _jkpart_body(h1_ref, h2_ref, l1_ref, o_ref):
    t = jnp.dot(h1_ref[...], l1_ref[0:_D], preferred_element_type=jnp.float32)
    t += jnp.dot(h2_ref[...], l1_ref[_D:2 * _D],
                 preferred_element_type=jnp.float32)
    o_ref[...] = t


def _jkpart_call(h1, h2, lin1_w):
    rows = pl.BlockSpec((_ROWS_BLK, _D), lambda i: (i, 0))
    return pl.pallas_call(
        _jkpart_body,
        grid=(_GRID,),
        in_specs=[rows, rows, pl.BlockSpec((3 * _D, _D), lambda i: (0, 0))],
        out_specs=rows,
        out_shape=jax.ShapeDtypeStruct((_N, _D), jnp.float32),
    )(h1, h2, lin1_w)


def _jkfin_body(x_ref, a0_ref, a1_ref, w1_ref, b1_ref, w2_ref, b2_ref,
                g_ref, be_ref, rm_ref, rv_ref, p_ref, l1_ref, jb1_ref,
                l2_ref, jb2_ref, o_ref):
    h = x_ref[...] + (a0_ref[0].astype(jnp.float32)
                      + a1_ref[0].astype(jnp.float32))
    t = jnp.dot(h, w1_ref[...], preferred_element_type=jnp.float32)
    t = jnp.maximum(t + b1_ref[...], 0.0)
    t = jnp.dot(t, w2_ref[...], preferred_element_type=jnp.float32)
    t = jnp.maximum(t + b2_ref[...], 0.0)
    inv = lax.rsqrt(rv_ref[...] + 1e-5)
    h3 = (t - rm_ref[...]) * inv * g_ref[...] + be_ref[...]
    t = p_ref[...] + jnp.dot(h3, l1_ref[2 * _D:3 * _D],
                             preferred_element_type=jnp.float32)
    t = jnp.maximum(t + jb1_ref[...], 0.0)
    o_ref[...] = jnp.dot(t, l2_ref[...],
                         preferred_element_type=jnp.float32) + jb2_ref[...]


def _jkfin_call(x, agg, w1, b1, w2, b2, g, be, rm, rv, p,
                lin1_w, lin1_b, lin2_w, lin2_b):
    rows = pl.BlockSpec((_ROWS_BLK, _D), lambda i: (i, 0))
    vec = pl.BlockSpec((1, _D), lambda i: (0, 0))
    return pl.pallas_call(
        _jkfin_body,
        grid=(_GRID,),
        in_specs=[
            rows,
            pl.BlockSpec((1, _ROWS_BLK, _D), lambda i: (0, i, 0)),
            pl.BlockSpec((1, _ROWS_BLK, _D), lambda i: (1, i, 0)),
            pl.BlockSpec((_D, _D), lambda i: (0, 0)),
            vec,
            pl.BlockSpec((_D, _D), lambda i: (0, 0)),
            vec, vec, vec, vec, vec,
            rows,
            pl.BlockSpec((3 * _D, _D), lambda i: (0, 0)),
            vec,
            pl.BlockSpec((_D, _OUT), lambda i: (0, 0)),
            pl.BlockSpec((1, _OUT), lambda i: (0, 0)),
        ],
        out_specs=pl.BlockSpec((_ROWS_BLK, _OUT), lambda i: (i, 0)),
        out_shape=jax.ShapeDtypeStruct((_N, _OUT), jnp.float32),
    )(x, agg, agg, w1, b1.reshape(1, _D), w2, b2.reshape(1, _D),
      g.reshape(1, _D), be.reshape(1, _D), rm.reshape(1, _D),
      rv.reshape(1, _D), p, lin1_w, lin1_b.reshape(1, _D), lin2_w,
      lin2_b.reshape(1, _OUT))


def kernel(x, edge_index, batch,
           w1_0, b1_0, w2_0, b2_0, g_0, be_0, rm_0, rv_0,
           w1_1, b1_1, w2_1, b2_1, g_1, be_1, rm_1, rv_1,
           w1_2, b1_2, w2_2, b2_2, g_2, be_2, rm_2, rv_2,
           lin1_w, lin1_b, lin2_w, lin2_b):
    del batch
    er = edge_index.reshape(2, _NW, _CHUNKS, _CHUNK)
    zero = jnp.zeros((_RPT, _D), jnp.bfloat16)

    params = [
        (w1_0, b1_0, w2_0, b2_0, g_0, be_0, rm_0, rv_0),
        (w1_1, b1_1, w2_1, b2_1, g_1, be_1, rm_1, rv_1),
        (w1_2, b1_2, w2_2, b2_2, g_2, be_2, rm_2, rv_2),
    ]
    h, h_bf = x, x.astype(jnp.bfloat16)
    hs = []
    for p in params[:2]:
        agg = _agg_call(h_bf, er, zero)
        h, h_bf = _mlp_call(h, agg, *p)
        hs.append(h)
    part = _jkpart_call(hs[0], hs[1], lin1_w)
    agg = _agg_call(h_bf, er, zero)
    return _jkfin_call(hs[1], agg, *params[2], part,
                       lin1_w, lin1_b, lin2_w, lin2_b)
